# Initial kernel scaffold; baseline (speedup 1.0000x reference)
import jax
import jax.numpy as jnp
from jax.experimental import pallas as pl


def kernel(entity, edge_index, emb_table, W1_l, b1, W1_r, W2_l, b2, W2_r, Wc, bc):
    raise NotImplementedError("write your pallas kernel here")


# speedup vs baseline: 5.0818x; 5.0818x over previous
import functools

import jax
import jax.numpy as jnp
from jax import lax
from jax.experimental import pallas as pl
from jax.experimental.pallas import tpu as pltpu
from jax.experimental.pallas import tpu_sc as plsc

N = 10000
E = 320000
EMBED_DIM = 128
HIDDEN_DIM = 128
NUM_NODE_TYPES = 16

D = 128
HALF = 64
NUM_SUBCORES = 16
CHUNK = 128
N_PAD = 10240
ROWS_PER_TILE = N_PAD // NUM_SUBCORES
EDGES_PER_TILE = 157 * CHUNK
E_PAD = EDGES_PER_TILE * NUM_SUBCORES
EMB_CHUNK = 64
EMB_ITERS = ROWS_PER_TILE // EMB_CHUNK
STAGE_ROWS = 160
STAGE_ITERS = ROWS_PER_TILE // STAGE_ROWS


def _make_sc_layer(first_layer: bool):
  mesh = plsc.VectorSubcoreMesh(core_axis_name="c", subcore_axis_name="s")

  if first_layer:
    out_type = (
        jax.ShapeDtypeStruct((N_PAD, D), jnp.float32),
        jax.ShapeDtypeStruct((N_PAD, D), jnp.float32),
        jax.ShapeDtypeStruct((N_PAD, 16), jnp.float32),
    )
  else:
    out_type = jax.ShapeDtypeStruct((N_PAD, D), jnp.float32)

  scratch_types = [
      pltpu.VMEM((EMB_CHUNK,), jnp.int32),
      pltpu.VMEM((EMB_CHUNK, D), jnp.float32),
      pltpu.VMEM((CHUNK,), jnp.int32),
      pltpu.VMEM((CHUNK,), jnp.int32),
      pltpu.VMEM((CHUNK, HALF), jnp.float32),
      pltpu.VMEM((CHUNK, 16), jnp.float32),
      pltpu.VMEM((STAGE_ROWS, HALF), jnp.float32),
      pltpu.VMEM((STAGE_ROWS, 16), jnp.float32),
      pltpu.VMEM_SHARED((N_PAD, HALF), jnp.float32),
      pltpu.VMEM_SHARED((N_PAD, HALF), jnp.float32),
      pltpu.VMEM_SHARED((N_PAD, 16), jnp.float32),
      pltpu.SemaphoreType.DMA,
  ]

  @functools.partial(
      pl.kernel, out_type=out_type, mesh=mesh, scratch_types=scratch_types,
      name="sc_sage_agg1" if first_layer else "sc_sage_agg2",
      compiler_params=pltpu.CompilerParams(use_tc_tiling_on_sc=False),
  )
  def sc_kernel(*refs):
    if first_layer:
      (entity_hbm, table_hbm, src_hbm, dst_hbm, zeros_hbm, ones_hbm,
       x_out, agg_out, cnt_out,
       eidx_v, emb_v, sidx_v, didx_v, msg_v, ones_v, stage_v, cstage_v,
       x_sh, agg_sh, cnt_sh, sem) = refs
    else:
      (h_hbm, src_hbm, dst_hbm, zeros_hbm, ones_hbm,
       agg_out,
       eidx_v, emb_v, sidx_v, didx_v, msg_v, ones_v, stage_v, cstage_v,
       x_sh, agg_sh, cnt_sh, sem) = refs

    c = lax.axis_index("c")
    s = lax.axis_index("s")
    row0 = s * ROWS_PER_TILE

    pltpu.sync_copy(zeros_hbm, stage_v)
    for k in range(STAGE_ITERS):
      pltpu.sync_copy(stage_v, agg_sh.at[pl.ds(row0 + k * STAGE_ROWS,
                                               STAGE_ROWS)])
    if first_layer:
      @pl.when(c == 0)
      def _():
        for k in range(STAGE_ITERS):
          pltpu.sync_copy(stage_v.at[:, pl.ds(0, 16)],
                          cnt_sh.at[pl.ds(row0 + k * STAGE_ROWS, STAGE_ROWS)])
      pltpu.sync_copy(ones_hbm, ones_v)

    if first_layer:
      for k in range(EMB_ITERS):
        base = row0 + k * EMB_CHUNK
        pltpu.sync_copy(entity_hbm.at[pl.ds(base, EMB_CHUNK)], eidx_v)
        pltpu.async_copy(table_hbm.at[eidx_v], emb_v, sem).wait()

        @pl.when(c == 0)
        def _():
          pltpu.sync_copy(emb_v, x_out.at[pl.ds(base, EMB_CHUNK)])

        pltpu.sync_copy(emb_v.at[:, pl.ds(c * HALF, HALF)],
                        x_sh.at[pl.ds(base, EMB_CHUNK)])
    else:
      for k in range(STAGE_ITERS):
        r = row0 + k * STAGE_ROWS
        pltpu.sync_copy(
            h_hbm.at[pl.ds(r, STAGE_ROWS), pl.ds(c * HALF, HALF)], stage_v)
        pltpu.sync_copy(stage_v, x_sh.at[pl.ds(r, STAGE_ROWS)])

    plsc.subcore_barrier()

    e0 = s * EDGES_PER_TILE

    def chunk_body(i, carry):
      ebase = e0 + i * CHUNK
      pltpu.sync_copy(src_hbm.at[pl.ds(ebase, CHUNK)], sidx_v)
      pltpu.sync_copy(dst_hbm.at[pl.ds(ebase, CHUNK)], didx_v)
      pltpu.async_copy(x_sh.at[sidx_v], msg_v, sem).wait()
      pltpu.sync_copy(msg_v, agg_sh.at[didx_v], add=True)
      if first_layer:
        @pl.when(c == 0)
        def _():
          pltpu.sync_copy(ones_v, cnt_sh.at[didx_v], add=True)
      return carry

    lax.fori_loop(0, EDGES_PER_TILE // CHUNK, chunk_body, 0)

    plsc.subcore_barrier()

    for k in range(STAGE_ITERS):
      r = row0 + k * STAGE_ROWS
      pltpu.sync_copy(agg_sh.at[pl.ds(r, STAGE_ROWS)], stage_v)
      pltpu.sync_copy(
          stage_v, agg_out.at[pl.ds(r, STAGE_ROWS), pl.ds(c * HALF, HALF)])
    if first_layer:
      @pl.when(c == 0)
      def _():
        for k in range(STAGE_ITERS):
          r = row0 + k * STAGE_ROWS
          pltpu.sync_copy(cnt_sh.at[pl.ds(r, STAGE_ROWS)], cstage_v)
          pltpu.sync_copy(cstage_v, cnt_out.at[pl.ds(r, STAGE_ROWS)])

  return sc_kernel


_sc_layer1 = _make_sc_layer(True)
_sc_layer2 = _make_sc_layer(False)


_ROW_BLK = 1024


def _dotT(a, w):
  return lax.dot_general(a, w, (((1,), (1,)), ((), ())),
                         precision=lax.Precision.HIGHEST,
                         preferred_element_type=jnp.float32)


def _tc1_body(x_ref, agg_ref, cnt_ref, wl_ref, b_ref, wr_ref, h_ref):
  inv = 1.0 / jnp.maximum(cnt_ref[:, 0:1], 1.0)
  mean = agg_ref[...] * inv
  h = _dotT(mean, wl_ref[...]) + b_ref[...] + _dotT(x_ref[...], wr_ref[...])
  h_ref[...] = jnp.maximum(h, 0.0)


def _tc2_body(h_ref, agg_ref, cnt_ref, wl_ref, b_ref, wr_ref, wc_ref, bc_ref,
              out_ref):
  inv = 1.0 / jnp.maximum(cnt_ref[:, 0:1], 1.0)
  mean = agg_ref[...] * inv
  h2 = _dotT(mean, wl_ref[...]) + b_ref[...] + _dotT(h_ref[...], wr_ref[...])
  out_ref[...] = _dotT(h2, wc_ref[...]) + bc_ref[...]


def _row_spec(width):
  return pl.BlockSpec((_ROW_BLK, width), lambda i: (i, 0))


def _full_spec(r, cdim):
  return pl.BlockSpec((r, cdim), lambda i: (0, 0))


def _tc_layer1(x, agg, cnt, W1_l, b1, W1_r):
  return pl.pallas_call(
      _tc1_body,
      grid=(N_PAD // _ROW_BLK,),
      in_specs=[
          _row_spec(D), _row_spec(D), _row_spec(16),
          _full_spec(D, D), _full_spec(1, D), _full_spec(D, D),
      ],
      out_specs=_row_spec(D),
      out_shape=jax.ShapeDtypeStruct((N_PAD, D), jnp.float32),
  )(x, agg, cnt, W1_l, b1.reshape(1, D), W1_r)


def _tc_layer2(h1, agg2, cnt, W2_l, b2, W2_r, Wc, bc):
  return pl.pallas_call(
      _tc2_body,
      grid=(N_PAD // _ROW_BLK,),
      in_specs=[
          _row_spec(D), _row_spec(D), _row_spec(16),
          _full_spec(D, D), _full_spec(1, D), _full_spec(D, D),
          _full_spec(NUM_NODE_TYPES, D), _full_spec(1, NUM_NODE_TYPES),
      ],
      out_specs=_row_spec(NUM_NODE_TYPES),
      out_shape=jax.ShapeDtypeStruct((N_PAD, NUM_NODE_TYPES), jnp.float32),
  )(h1, agg2, cnt, W2_l, b2.reshape(1, D), W2_r, Wc,
    bc.reshape(1, NUM_NODE_TYPES))


@jax.jit
def kernel(entity, edge_index, emb_table, W1_l, b1, W1_r, W2_l, b2, W2_r, Wc,
           bc):
  entity_pad = jnp.pad(entity.astype(jnp.int32), (0, N_PAD - N))
  src = edge_index[0].astype(jnp.int32)
  dst = edge_index[1].astype(jnp.int32)
  src_pad = jnp.pad(src, (0, E_PAD - E))
  dst_pad = jnp.pad(dst, (0, E_PAD - E), constant_values=N)
  zeros = jnp.zeros((STAGE_ROWS, HALF), jnp.float32)
  ones = jnp.ones((CHUNK, 16), jnp.float32)

  x, agg1, cnt = _sc_layer1(entity_pad, emb_table, src_pad, dst_pad, zeros,
                            ones)
  h1 = _tc_layer1(x, agg1, cnt, W1_l, b1, W1_r)
  agg2 = _sc_layer2(h1, src_pad, dst_pad, zeros, ones)
  out = _tc_layer2(h1, agg2, cnt, W2_l, b2, W2_r, Wc, bc)
  return out[:N]


# speedup vs baseline: 7.5515x; 1.4860x over previous
import functools

import jax
import jax.numpy as jnp
from jax import lax
from jax.experimental import pallas as pl
from jax.experimental.pallas import tpu as pltpu
from jax.experimental.pallas import tpu_sc as plsc

N = 10000
E = 320000
EMBED_DIM = 128
HIDDEN_DIM = 128
NUM_NODE_TYPES = 16

D = 128
HALF = 64
NUM_SUBCORES = 16
CHUNK = 128
N_PAD = 10240
ROWS_PER_TILE = N_PAD // NUM_SUBCORES
BLK = 16
NBLK = 10
EDGES_PER_TILE = NBLK * BLK * CHUNK
E_PAD = EDGES_PER_TILE * NUM_SUBCORES
EMB_CHUNK = 32
EMB_ITERS = ROWS_PER_TILE // EMB_CHUNK
STAGE_ROWS = 128
STAGE_ITERS = ROWS_PER_TILE // STAGE_ROWS


def _make_sc_layer(first_layer: bool):
  mesh = plsc.VectorSubcoreMesh(core_axis_name="c", subcore_axis_name="s")

  if first_layer:
    out_type = (
        jax.ShapeDtypeStruct((N_PAD, D), jnp.float32),
        jax.ShapeDtypeStruct((N_PAD, D), jnp.float32),
        jax.ShapeDtypeStruct((N_PAD, 16), jnp.float32),
    )
  else:
    out_type = jax.ShapeDtypeStruct((N_PAD, D), jnp.float32)

  scratch_types = [
      pltpu.VMEM((EMB_CHUNK,), jnp.int32),
      pltpu.VMEM((EMB_CHUNK, D), jnp.float32),
      pltpu.VMEM((BLK, CHUNK), jnp.int32),
      pltpu.VMEM((BLK, CHUNK), jnp.int32),
      pltpu.VMEM((CHUNK, HALF), jnp.float32),
      pltpu.VMEM((CHUNK, HALF), jnp.float32),
      pltpu.VMEM((CHUNK, 16), jnp.float32),
      pltpu.VMEM((STAGE_ROWS, HALF), jnp.float32),
      pltpu.VMEM((STAGE_ROWS, 16), jnp.float32),
      pltpu.VMEM_SHARED((N_PAD, HALF), jnp.float32),
      pltpu.VMEM_SHARED((N_PAD, HALF), jnp.float32),
      pltpu.VMEM_SHARED((N_PAD, 16), jnp.float32),
      pltpu.SemaphoreType.DMA,
      pltpu.SemaphoreType.DMA,
      pltpu.SemaphoreType.DMA,
      pltpu.SemaphoreType.DMA,
      pltpu.SemaphoreType.DMA,
  ]

  @functools.partial(
      pl.kernel, out_type=out_type, mesh=mesh, scratch_types=scratch_types,
      name="sc_sage_agg1" if first_layer else "sc_sage_agg2",
      compiler_params=pltpu.CompilerParams(use_tc_tiling_on_sc=False),
  )
  def sc_kernel(*refs):
    if first_layer:
      (entity_hbm, table_hbm, src_hbm, dst_hbm, zeros_hbm, ones_hbm,
       x_out, agg_out, cnt_out,
       eidx_v, emb_v, sidx_v, didx_v, msg0_v, msg1_v, ones_v, stage_v,
       cstage_v, x_sh, agg_sh, cnt_sh,
       sem, sem_g0, sem_g1, sem_s0, sem_s1) = refs
    else:
      (h_hbm, src_hbm, dst_hbm, zeros_hbm, ones_hbm,
       agg_out,
       eidx_v, emb_v, sidx_v, didx_v, msg0_v, msg1_v, ones_v, stage_v,
       cstage_v, x_sh, agg_sh, cnt_sh,
       sem, sem_g0, sem_g1, sem_s0, sem_s1) = refs

    c = lax.axis_index("c")
    s = lax.axis_index("s")
    row0 = s * ROWS_PER_TILE

    pltpu.sync_copy(zeros_hbm, stage_v)
    for k in range(STAGE_ITERS):
      pltpu.sync_copy(stage_v, agg_sh.at[pl.ds(row0 + k * STAGE_ROWS,
                                               STAGE_ROWS)])
    if first_layer:
      @pl.when(c == 0)
      def _():
        for k in range(STAGE_ITERS):
          pltpu.sync_copy(stage_v.at[:, pl.ds(0, 16)],
                          cnt_sh.at[pl.ds(row0 + k * STAGE_ROWS, STAGE_ROWS)])
      pltpu.sync_copy(ones_hbm, ones_v)

    if first_layer:
      for k in range(EMB_ITERS):
        base = row0 + k * EMB_CHUNK
        pltpu.sync_copy(entity_hbm.at[pl.ds(base, EMB_CHUNK)], eidx_v)
        pltpu.async_copy(table_hbm.at[eidx_v], emb_v, sem).wait()

        @pl.when(c == 0)
        def _():
          pltpu.sync_copy(emb_v, x_out.at[pl.ds(base, EMB_CHUNK)])

        pltpu.sync_copy(emb_v.at[:, pl.ds(c * HALF, HALF)],
                        x_sh.at[pl.ds(base, EMB_CHUNK)])
    else:
      for k in range(STAGE_ITERS):
        r = row0 + k * STAGE_ROWS
        pltpu.sync_copy(
            h_hbm.at[pl.ds(r, STAGE_ROWS), pl.ds(c * HALF, HALF)], stage_v)
        pltpu.sync_copy(stage_v, x_sh.at[pl.ds(r, STAGE_ROWS)])

    plsc.subcore_barrier()

    e0 = s * EDGES_PER_TILE
    msgs = (msg0_v, msg1_v)
    gsems = (sem_g0, sem_g1)
    ssems = (sem_s0, sem_s1)

    def block_body(blk, carry):
      brow = s * (NBLK * BLK) + blk * BLK
      pltpu.sync_copy(src_hbm.at[pl.ds(brow, BLK)], sidx_v)
      pltpu.sync_copy(dst_hbm.at[pl.ds(brow, BLK)], didx_v)
      cnt_copies = []
      gathers = [None, None]
      scatters = [None, None]
      gathers[0] = pltpu.async_copy(x_sh.at[sidx_v.at[0]], msgs[0], gsems[0])
      for j in range(BLK):
        p = j % 2
        q = (j + 1) % 2
        if j + 1 < BLK:
          if scatters[q] is not None:
            scatters[q].wait()
            scatters[q] = None
          gathers[q] = pltpu.async_copy(x_sh.at[sidx_v.at[j + 1]], msgs[q],
                                        gsems[q])
        gathers[p].wait()
        scatters[p] = pltpu.async_copy(msgs[p], agg_sh.at[didx_v.at[j]],
                                       ssems[p], add=True)
        if first_layer:
          @pl.when(c == 0)
          def _():
            cnt_copies.append(
                pltpu.async_copy(ones_v, cnt_sh.at[didx_v.at[j]], sem,
                                 add=True))
      for sc in scatters:
        if sc is not None:
          sc.wait()
      if first_layer:
        @pl.when(c == 0)
        def _():
          for cp in cnt_copies:
            cp.wait()
      return carry

    lax.fori_loop(0, NBLK, block_body, 0)

    plsc.subcore_barrier()

    for k in range(STAGE_ITERS):
      r = row0 + k * STAGE_ROWS
      pltpu.sync_copy(agg_sh.at[pl.ds(r, STAGE_ROWS)], stage_v)
      pltpu.sync_copy(
          stage_v, agg_out.at[pl.ds(r, STAGE_ROWS), pl.ds(c * HALF, HALF)])
    if first_layer:
      @pl.when(c == 0)
      def _():
        for k in range(STAGE_ITERS):
          r = row0 + k * STAGE_ROWS
          pltpu.sync_copy(cnt_sh.at[pl.ds(r, STAGE_ROWS)], cstage_v)
          pltpu.sync_copy(cstage_v, cnt_out.at[pl.ds(r, STAGE_ROWS)])

  return sc_kernel


_sc_layer1 = _make_sc_layer(True)
_sc_layer2 = _make_sc_layer(False)


_ROW_BLK = 1024


def _dotT(a, w):
  return lax.dot_general(a, w, (((1,), (1,)), ((), ())),
                         precision=lax.Precision.HIGHEST,
                         preferred_element_type=jnp.float32)


def _tc1_body(x_ref, agg_ref, cnt_ref, wl_ref, b_ref, wr_ref, h_ref):
  inv = 1.0 / jnp.maximum(cnt_ref[:, 0:1], 1.0)
  mean = agg_ref[...] * inv
  h = _dotT(mean, wl_ref[...]) + b_ref[...] + _dotT(x_ref[...], wr_ref[...])
  h_ref[...] = jnp.maximum(h, 0.0)


def _tc2_body(h_ref, agg_ref, cnt_ref, wl_ref, b_ref, wr_ref, wc_ref, bc_ref,
              out_ref):
  inv = 1.0 / jnp.maximum(cnt_ref[:, 0:1], 1.0)
  mean = agg_ref[...] * inv
  h2 = _dotT(mean, wl_ref[...]) + b_ref[...] + _dotT(h_ref[...], wr_ref[...])
  out_ref[...] = _dotT(h2, wc_ref[...]) + bc_ref[...]


def _row_spec(width):
  return pl.BlockSpec((_ROW_BLK, width), lambda i: (i, 0))


def _full_spec(r, cdim):
  return pl.BlockSpec((r, cdim), lambda i: (0, 0))


def _tc_layer1(x, agg, cnt, W1_l, b1, W1_r):
  return pl.pallas_call(
      _tc1_body,
      grid=(N_PAD // _ROW_BLK,),
      in_specs=[
          _row_spec(D), _row_spec(D), _row_spec(16),
          _full_spec(D, D), _full_spec(1, D), _full_spec(D, D),
      ],
      out_specs=_row_spec(D),
      out_shape=jax.ShapeDtypeStruct((N_PAD, D), jnp.float32),
  )(x, agg, cnt, W1_l, b1.reshape(1, D), W1_r)


def _tc_layer2(h1, agg2, cnt, W2_l, b2, W2_r, Wc, bc):
  return pl.pallas_call(
      _tc2_body,
      grid=(N_PAD // _ROW_BLK,),
      in_specs=[
          _row_spec(D), _row_spec(D), _row_spec(16),
          _full_spec(D, D), _full_spec(1, D), _full_spec(D, D),
          _full_spec(NUM_NODE_TYPES, D), _full_spec(1, NUM_NODE_TYPES),
      ],
      out_specs=_row_spec(NUM_NODE_TYPES),
      out_shape=jax.ShapeDtypeStruct((N_PAD, NUM_NODE_TYPES), jnp.float32),
  )(h1, agg2, cnt, W2_l, b2.reshape(1, D), W2_r, Wc,
    bc.reshape(1, NUM_NODE_TYPES))


@jax.jit
def kernel(entity, edge_index, emb_table, W1_l, b1, W1_r, W2_l, b2, W2_r, Wc,
           bc):
  entity_pad = jnp.pad(entity.astype(jnp.int32), (0, N_PAD - N))
  src = edge_index[0].astype(jnp.int32)
  dst = edge_index[1].astype(jnp.int32)
  src_pad = jnp.pad(src, (0, E_PAD - E)).reshape(E_PAD // CHUNK, CHUNK)
  dst_pad = jnp.pad(dst, (0, E_PAD - E),
                    constant_values=N).reshape(E_PAD // CHUNK, CHUNK)
  zeros = jnp.zeros((STAGE_ROWS, HALF), jnp.float32)
  ones = jnp.ones((CHUNK, 16), jnp.float32)

  x, agg1, cnt = _sc_layer1(entity_pad, emb_table, src_pad, dst_pad, zeros,
                            ones)
  h1 = _tc_layer1(x, agg1, cnt, W1_l, b1, W1_r)
  agg2 = _sc_layer2(h1, src_pad, dst_pad, zeros, ones)
  out = _tc_layer2(h1, agg2, cnt, W2_l, b2, W2_r, Wc, bc)
  return out[:N]


# speedup vs baseline: 7.6367x; 1.0113x over previous
import functools

import jax
import jax.numpy as jnp
from jax import lax
from jax.experimental import pallas as pl
from jax.experimental.pallas import tpu as pltpu
from jax.experimental.pallas import tpu_sc as plsc

N = 10000
E = 320000
EMBED_DIM = 128
HIDDEN_DIM = 128
NUM_NODE_TYPES = 16

D = 128
HALF = 64
NUM_SUBCORES = 16
CHUNK = 128
N_PAD = 10240
ROWS_PER_TILE = N_PAD // NUM_SUBCORES
BLK = 16
NBLK = 10
EDGES_PER_TILE = NBLK * BLK * CHUNK
E_PAD = EDGES_PER_TILE * NUM_SUBCORES
EMB_CHUNK = 16
EMB_ITERS = ROWS_PER_TILE // EMB_CHUNK
STAGE_ROWS = 64
STAGE_ITERS = ROWS_PER_TILE // STAGE_ROWS


def _make_sc_layer(first_layer: bool):
  mesh = plsc.VectorSubcoreMesh(core_axis_name="c", subcore_axis_name="s")

  if first_layer:
    out_type = (
        jax.ShapeDtypeStruct((N_PAD, D), jnp.float32),
        jax.ShapeDtypeStruct((N_PAD, D), jnp.float32),
        jax.ShapeDtypeStruct((N_PAD, 16), jnp.float32),
    )
  else:
    out_type = jax.ShapeDtypeStruct((N_PAD, D), jnp.float32)

  scratch_types = [
      pltpu.VMEM((EMB_CHUNK,), jnp.int32),
      pltpu.VMEM((EMB_CHUNK,), jnp.int32),
      pltpu.VMEM((EMB_CHUNK, D), jnp.float32),
      pltpu.VMEM((EMB_CHUNK, D), jnp.float32),
      pltpu.VMEM((BLK, CHUNK), jnp.int32),
      pltpu.VMEM((BLK, CHUNK), jnp.int32),
      pltpu.VMEM((CHUNK, HALF), jnp.float32),
      pltpu.VMEM((CHUNK, HALF), jnp.float32),
      pltpu.VMEM((STAGE_ROWS, HALF), jnp.float32),
      pltpu.VMEM((STAGE_ROWS, HALF), jnp.float32),
      pltpu.VMEM((CHUNK, 16), jnp.float32),
      pltpu.VMEM_SHARED((N_PAD, HALF), jnp.float32),
      pltpu.VMEM_SHARED((N_PAD, HALF), jnp.float32),
      pltpu.VMEM_SHARED((N_PAD, 16), jnp.float32),
      pltpu.SemaphoreType.DMA,
      pltpu.SemaphoreType.DMA,
      pltpu.SemaphoreType.DMA,
      pltpu.SemaphoreType.DMA,
      pltpu.SemaphoreType.DMA,
      pltpu.SemaphoreType.DMA,
      pltpu.SemaphoreType.DMA,
  ]

  @functools.partial(
      pl.kernel, out_type=out_type, mesh=mesh, scratch_types=scratch_types,
      name="sc_sage_agg1" if first_layer else "sc_sage_agg2",
      compiler_params=pltpu.CompilerParams(use_tc_tiling_on_sc=False),
  )
  def sc_kernel(*refs):
    if first_layer:
      (entity_hbm, table_hbm, src_hbm, dst_hbm, zeros_hbm, ones_hbm,
       x_out, agg_out, cnt_out, *rest) = refs
      h_hbm = None
    else:
      (h_hbm, src_hbm, dst_hbm, zeros_hbm, ones_hbm, agg_out, *rest) = refs
    (eidx0, eidx1, emb0, emb1, sidx_v, didx_v, msg0_v, msg1_v,
     stage0, stage1, cstage_v, x_sh, agg_sh, cnt_sh,
     sem, sem_g0, sem_g1, sem_s0, sem_s1, sem_w0, sem_w1) = rest
    eidxs = (eidx0, eidx1)
    embs = (emb0, emb1)
    msgs = (msg0_v, msg1_v)
    stages = (stage0, stage1)
    gsems = (sem_g0, sem_g1)
    ssems = (sem_s0, sem_s1)
    wsems = (sem_w0, sem_w1)

    c = lax.axis_index("c")
    s = lax.axis_index("s")
    row0 = s * ROWS_PER_TILE
    chalf = c * HALF

    pltpu.sync_copy(zeros_hbm, stage0)
    zcp = []
    for k in range(STAGE_ITERS):
      zcp.append(pltpu.async_copy(
          stage0, agg_sh.at[pl.ds(row0 + k * STAGE_ROWS, STAGE_ROWS)],
          sem_w0))
    if first_layer:
      @pl.when(c == 1)
      def _():
        zc = [pltpu.async_copy(
            stage0.at[:, pl.ds(0, 16)],
            cnt_sh.at[pl.ds(row0 + k * STAGE_ROWS, STAGE_ROWS)], sem_w1)
            for k in range(STAGE_ITERS)]
        for cp in zc:
          cp.wait()
      pltpu.sync_copy(ones_hbm, cstage_v)
    for cp in zcp:
      cp.wait()

    if first_layer:
      gth = [None, None]
      wrs = [[], []]
      pltpu.sync_copy(entity_hbm.at[pl.ds(row0, EMB_CHUNK)], eidxs[0])
      gth[0] = pltpu.async_copy(table_hbm.at[eidxs[0]], embs[0], gsems[0])
      for k in range(EMB_ITERS):
        p = k % 2
        q = (k + 1) % 2
        base = row0 + k * EMB_CHUNK
        if k + 1 < EMB_ITERS:
          for w in wrs[q]:
            w.wait()
          wrs[q] = []
          nbase = base + EMB_CHUNK
          pltpu.sync_copy(entity_hbm.at[pl.ds(nbase, EMB_CHUNK)], eidxs[q])
          gth[q] = pltpu.async_copy(table_hbm.at[eidxs[q]], embs[q], gsems[q])
        gth[p].wait()
        half = embs[p].at[:, pl.ds(chalf, HALF)]
        wrs[p] = [
            pltpu.async_copy(half,
                             x_out.at[pl.ds(base, EMB_CHUNK),
                                      pl.ds(chalf, HALF)], wsems[p]),
            pltpu.async_copy(half, x_sh.at[pl.ds(base, EMB_CHUNK)], ssems[p]),
        ]
      for ws in wrs:
        for w in ws:
          w.wait()
    else:
      stg = [None, None]
      for k in range(STAGE_ITERS):
        p = k % 2
        r = row0 + k * STAGE_ROWS
        if stg[p] is not None:
          stg[p].wait()
        pltpu.sync_copy(
            h_hbm.at[pl.ds(r, STAGE_ROWS), pl.ds(chalf, HALF)], stages[p])
        stg[p] = pltpu.async_copy(stages[p], x_sh.at[pl.ds(r, STAGE_ROWS)],
                                  ssems[p])
      for st in stg:
        if st is not None:
          st.wait()

    plsc.subcore_barrier()

    def block_body(blk, carry):
      brow = s * (NBLK * BLK) + blk * BLK
      pltpu.sync_copy(src_hbm.at[pl.ds(brow, BLK)], sidx_v)
      pltpu.sync_copy(dst_hbm.at[pl.ds(brow, BLK)], didx_v)
      cnt_copies = []
      gathers = [None, None]
      scatters = [None, None]
      gathers[0] = pltpu.async_copy(x_sh.at[sidx_v.at[0]], msgs[0], gsems[0])
      for j in range(BLK):
        p = j % 2
        q = (j + 1) % 2
        if j + 1 < BLK:
          if scatters[q] is not None:
            scatters[q].wait()
            scatters[q] = None
          gathers[q] = pltpu.async_copy(x_sh.at[sidx_v.at[j + 1]], msgs[q],
                                        gsems[q])
        gathers[p].wait()
        scatters[p] = pltpu.async_copy(msgs[p], agg_sh.at[didx_v.at[j]],
                                       ssems[p], add=True)
        if first_layer:
          @pl.when(c == 1)
          def _():
            cnt_copies.append(
                pltpu.async_copy(cstage_v, cnt_sh.at[didx_v.at[j]], sem,
                                 add=True))
      for sc in scatters:
        if sc is not None:
          sc.wait()
      if first_layer:
        @pl.when(c == 1)
        def _():
          for cp in cnt_copies:
            cp.wait()
      return carry

    lax.fori_loop(0, NBLK, block_body, 0)

    plsc.subcore_barrier()

    wr = [None, None]
    for k in range(STAGE_ITERS):
      p = k % 2
      r = row0 + k * STAGE_ROWS
      if wr[p] is not None:
        wr[p].wait()
      pltpu.sync_copy(agg_sh.at[pl.ds(r, STAGE_ROWS)], stages[p])
      wr[p] = pltpu.async_copy(
          stages[p], agg_out.at[pl.ds(r, STAGE_ROWS), pl.ds(chalf, HALF)],
          wsems[p])
    for w in wr:
      if w is not None:
        w.wait()
    if first_layer:
      @pl.when(c == 1)
      def _():
        for k in range(ROWS_PER_TILE // CHUNK):
          r = row0 + k * CHUNK
          pltpu.sync_copy(cnt_sh.at[pl.ds(r, CHUNK)], cstage_v)
          pltpu.sync_copy(cstage_v, cnt_out.at[pl.ds(r, CHUNK)])

  return sc_kernel


_sc_layer1 = _make_sc_layer(True)
_sc_layer2 = _make_sc_layer(False)


_ROW_BLK = 1024


def _dotT(a, w):
  return lax.dot_general(a, w, (((1,), (1,)), ((), ())),
                         precision=lax.Precision.HIGHEST,
                         preferred_element_type=jnp.float32)


def _tc1_body(x_ref, agg_ref, cnt_ref, wl_ref, b_ref, wr_ref, h_ref):
  inv = 1.0 / jnp.maximum(cnt_ref[:, 0:1], 1.0)
  mean = agg_ref[...] * inv
  h = _dotT(mean, wl_ref[...]) + b_ref[...] + _dotT(x_ref[...], wr_ref[...])
  h_ref[...] = jnp.maximum(h, 0.0)


def _tc2_body(h_ref, agg_ref, cnt_ref, wl_ref, b_ref, wr_ref, wc_ref, bc_ref,
              out_ref):
  inv = 1.0 / jnp.maximum(cnt_ref[:, 0:1], 1.0)
  mean = agg_ref[...] * inv
  h2 = _dotT(mean, wl_ref[...]) + b_ref[...] + _dotT(h_ref[...], wr_ref[...])
  out_ref[...] = _dotT(h2, wc_ref[...]) + bc_ref[...]


def _row_spec(width):
  return pl.BlockSpec((_ROW_BLK, width), lambda i: (i, 0))


def _full_spec(r, cdim):
  return pl.BlockSpec((r, cdim), lambda i: (0, 0))


def _tc_layer1(x, agg, cnt, W1_l, b1, W1_r):
  return pl.pallas_call(
      _tc1_body,
      grid=(N_PAD // _ROW_BLK,),
      in_specs=[
          _row_spec(D), _row_spec(D), _row_spec(16),
          _full_spec(D, D), _full_spec(1, D), _full_spec(D, D),
      ],
      out_specs=_row_spec(D),
      out_shape=jax.ShapeDtypeStruct((N_PAD, D), jnp.float32),
  )(x, agg, cnt, W1_l, b1.reshape(1, D), W1_r)


def _tc_layer2(h1, agg2, cnt, W2_l, b2, W2_r, Wc, bc):
  return pl.pallas_call(
      _tc2_body,
      grid=(N_PAD // _ROW_BLK,),
      in_specs=[
          _row_spec(D), _row_spec(D), _row_spec(16),
          _full_spec(D, D), _full_spec(1, D), _full_spec(D, D),
          _full_spec(NUM_NODE_TYPES, D), _full_spec(1, NUM_NODE_TYPES),
      ],
      out_specs=_row_spec(NUM_NODE_TYPES),
      out_shape=jax.ShapeDtypeStruct((N_PAD, NUM_NODE_TYPES), jnp.float32),
  )(h1, agg2, cnt, W2_l, b2.reshape(1, D), W2_r, Wc,
    bc.reshape(1, NUM_NODE_TYPES))


@jax.jit
def kernel(entity, edge_index, emb_table, W1_l, b1, W1_r, W2_l, b2, W2_r, Wc,
           bc):
  entity_pad = jnp.pad(entity.astype(jnp.int32), (0, N_PAD - N))
  src = edge_index[0].astype(jnp.int32)
  dst = edge_index[1].astype(jnp.int32)
  src_pad = jnp.pad(src, (0, E_PAD - E)).reshape(E_PAD // CHUNK, CHUNK)
  dst_pad = jnp.pad(dst, (0, E_PAD - E),
                    constant_values=N).reshape(E_PAD // CHUNK, CHUNK)
  zeros = jnp.zeros((STAGE_ROWS, HALF), jnp.float32)
  ones = jnp.ones((CHUNK, 16), jnp.float32)

  x, agg1, cnt = _sc_layer1(entity_pad, emb_table, src_pad, dst_pad, zeros,
                            ones)
  h1 = _tc_layer1(x, agg1, cnt, W1_l, b1, W1_r)
  agg2 = _sc_layer2(h1, src_pad, dst_pad, zeros, ones)
  out = _tc_layer2(h1, agg2, cnt, W2_l, b2, W2_r, Wc, bc)
  return out[:N]


# speedup vs baseline: 8.9817x; 1.1761x over previous
import functools

import jax
import jax.numpy as jnp
from jax import lax
from jax.experimental import pallas as pl
from jax.experimental.pallas import tpu as pltpu
from jax.experimental.pallas import tpu_sc as plsc

N = 10000
E = 320000
NUM_NODE_TYPES = 16

D = 128
HALF = 64
NUM_SUBCORES = 16
CHUNK = 128
BLK = 32
NBLK = 5
N_PAD = 10240
ROWS_PER_TILE = N_PAD // NUM_SUBCORES
EDGES_PER_TILE = NBLK * BLK * CHUNK
E_PAD = EDGES_PER_TILE * NUM_SUBCORES
EMB_CHUNK = 64
EMB_ITERS = ROWS_PER_TILE // EMB_CHUNK
STAGE_ROWS = CHUNK
STAGE_ITERS = ROWS_PER_TILE // STAGE_ROWS


def _make_sc_layer(first_layer: bool):
  mesh = plsc.VectorSubcoreMesh(core_axis_name="c", subcore_axis_name="s")

  if first_layer:
    out_type = (
        jax.ShapeDtypeStruct((N_PAD, D), jnp.bfloat16),
        jax.ShapeDtypeStruct((N_PAD, D), jnp.bfloat16),
        jax.ShapeDtypeStruct((N_PAD, 16), jnp.float32),
        jax.ShapeDtypeStruct((N_PAD, 16), jnp.float32),
    )
  else:
    out_type = jax.ShapeDtypeStruct((N_PAD, D), jnp.bfloat16)

  scratch_types = [
      pltpu.VMEM((EMB_CHUNK,), jnp.int32),
      pltpu.VMEM((EMB_CHUNK,), jnp.int32),
      pltpu.VMEM((EMB_CHUNK, D), jnp.bfloat16),
      pltpu.VMEM((EMB_CHUNK, D), jnp.bfloat16),
      pltpu.VMEM((BLK, CHUNK), jnp.int32),
      pltpu.VMEM((BLK, CHUNK), jnp.int32),
      pltpu.VMEM((CHUNK, HALF), jnp.bfloat16),
      pltpu.VMEM((CHUNK, HALF), jnp.bfloat16),
      pltpu.VMEM((CHUNK, HALF), jnp.bfloat16),
      pltpu.VMEM((CHUNK, 16), jnp.float32),
      pltpu.VMEM_SHARED((N_PAD, HALF), jnp.bfloat16),
      pltpu.VMEM_SHARED((N_PAD, HALF), jnp.bfloat16),
      pltpu.VMEM_SHARED((N_PAD, 16), jnp.float32),
      pltpu.SemaphoreType.DMA,
      pltpu.SemaphoreType.DMA,
      pltpu.SemaphoreType.DMA,
      pltpu.SemaphoreType.DMA,
      pltpu.SemaphoreType.DMA,
      pltpu.SemaphoreType.DMA,
      pltpu.SemaphoreType.DMA,
      pltpu.SemaphoreType.DMA,
      pltpu.SemaphoreType.DMA,
  ]

  @functools.partial(
      pl.kernel, out_type=out_type, mesh=mesh, scratch_types=scratch_types,
      name="sc_sage_agg1" if first_layer else "sc_sage_agg2",
      compiler_params=pltpu.CompilerParams(use_tc_tiling_on_sc=False),
  )
  def sc_kernel(*refs):
    if first_layer:
      (entity_hbm, table_hbm, src_hbm, dst_hbm, zeros_hbm, ones_hbm,
       x_out, agg_out, cnt0_out, cnt1_out, *rest) = refs
      h_hbm = None
    else:
      (h_hbm, src_hbm, dst_hbm, zeros_hbm, ones_hbm, agg_out, *rest) = refs
      cnt0_out = cnt1_out = None
    (eidx0, eidx1, emb0, emb1, sidx_v, didx_v, msg0_v, msg1_v, msg2_v,
     cstage_v, x_sh, agg_sh, cnt_sh,
     sem, sem_g0, sem_g1, sem_g2, sem_s0, sem_s1, sem_s2, sem_w0,
     sem_w1) = rest
    eidxs = (eidx0, eidx1)
    embs = (emb0, emb1)
    msgs = (msg0_v, msg1_v, msg2_v)
    gsems = (sem_g0, sem_g1, sem_g2)
    ssems = (sem_s0, sem_s1, sem_s2)
    wsems = (sem_w0, sem_w1)

    c = lax.axis_index("c")
    s = lax.axis_index("s")
    row0 = s * ROWS_PER_TILE
    chalf = c * HALF

    pltpu.sync_copy(zeros_hbm, msgs[0])
    zcp = []
    for k in range(STAGE_ITERS):
      r = row0 + k * STAGE_ROWS
      zcp.append(pltpu.async_copy(msgs[0], agg_sh.at[pl.ds(r, STAGE_ROWS)],
                                  sem_w0))
    if first_layer:
      pltpu.sync_copy(ones_hbm.at[pl.ds(0, CHUNK)], cstage_v)
      for k in range(STAGE_ITERS):
        r = row0 + k * STAGE_ROWS
        pltpu.sync_copy(cstage_v, cnt_sh.at[pl.ds(r, STAGE_ROWS)])
      pltpu.sync_copy(ones_hbm.at[pl.ds(CHUNK, CHUNK)], cstage_v)
    for cp in zcp:
      cp.wait()

    if first_layer:
      gth = [None, None]
      wrs = [[], []]
      pltpu.sync_copy(entity_hbm.at[pl.ds(row0, EMB_CHUNK)], eidxs[0])
      gth[0] = pltpu.async_copy(table_hbm.at[eidxs[0]], embs[0], gsems[0])
      for k in range(EMB_ITERS):
        p = k % 2
        q = (k + 1) % 2
        base = row0 + k * EMB_CHUNK
        if k + 1 < EMB_ITERS:
          for w in wrs[q]:
            w.wait()
          wrs[q] = []
          nbase = base + EMB_CHUNK
          pltpu.sync_copy(entity_hbm.at[pl.ds(nbase, EMB_CHUNK)], eidxs[q])
          gth[q] = pltpu.async_copy(table_hbm.at[eidxs[q]], embs[q], gsems[q])
        gth[p].wait()
        half = embs[p].at[:, pl.ds(chalf, HALF)]
        wrs[p] = [
            pltpu.async_copy(half,
                             x_out.at[pl.ds(base, EMB_CHUNK),
                                      pl.ds(chalf, HALF)], wsems[p]),
            pltpu.async_copy(half, x_sh.at[pl.ds(base, EMB_CHUNK)], ssems[p]),
        ]
      for ws in wrs:
        for w in ws:
          w.wait()
    else:
      stg = [None, None]
      for k in range(STAGE_ITERS):
        p = k % 2
        r = row0 + k * STAGE_ROWS
        if stg[p] is not None:
          stg[p].wait()
        pltpu.sync_copy(
            h_hbm.at[pl.ds(r, STAGE_ROWS), pl.ds(chalf, HALF)], msgs[p])
        stg[p] = pltpu.async_copy(msgs[p], x_sh.at[pl.ds(r, STAGE_ROWS)],
                                  ssems[p])
      for st in stg:
        if st is not None:
          st.wait()

    plsc.subcore_barrier()

    def block_body(blk, carry):
      brow = s * (NBLK * BLK) + blk * BLK
      pltpu.sync_copy(src_hbm.at[pl.ds(brow, BLK)], sidx_v)
      pltpu.sync_copy(dst_hbm.at[pl.ds(brow, BLK)], didx_v)
      cnt_copies = []
      gathers = [None, None, None]
      scatters = [None, None, None]
      gathers[0] = pltpu.async_copy(x_sh.at[sidx_v.at[0]], msgs[0], gsems[0])
      gathers[1] = pltpu.async_copy(x_sh.at[sidx_v.at[1]], msgs[1], gsems[1])
      for j in range(BLK):
        p = j % 3
        q = (j + 2) % 3
        if j + 2 < BLK:
          if scatters[q] is not None:
            scatters[q].wait()
            scatters[q] = None
          gathers[q] = pltpu.async_copy(x_sh.at[sidx_v.at[j + 2]], msgs[q],
                                        gsems[q])
        gathers[p].wait()
        scatters[p] = pltpu.async_copy(msgs[p], agg_sh.at[didx_v.at[j]],
                                       ssems[p], add=True)
        if first_layer:
          count_here = (c == 0) if j < BLK // 2 else (c == 1)

          @pl.when(count_here)
          def _():
            cnt_copies.append(
                pltpu.async_copy(cstage_v, cnt_sh.at[didx_v.at[j]], sem,
                                 add=True))
      for sc in scatters:
        if sc is not None:
          sc.wait()
      if first_layer:
        for j, cp in enumerate(cnt_copies):
          count_here = (c == 0) if j < BLK // 2 else (c == 1)

          @pl.when(count_here)
          def _():
            cp.wait()
      return carry

    lax.fori_loop(0, NBLK, block_body, 0)

    plsc.subcore_barrier()

    wr = [None, None]
    for k in range(STAGE_ITERS):
      p = k % 2
      r = row0 + k * STAGE_ROWS
      if wr[p] is not None:
        wr[p].wait()
      pltpu.sync_copy(agg_sh.at[pl.ds(r, STAGE_ROWS)], msgs[p])
      wr[p] = pltpu.async_copy(
          msgs[p], agg_out.at[pl.ds(r, STAGE_ROWS), pl.ds(chalf, HALF)],
          wsems[p])
    for w in wr:
      if w is not None:
        w.wait()
    if first_layer:
      for k in range(ROWS_PER_TILE // CHUNK):
        r = row0 + k * CHUNK
        pltpu.sync_copy(cnt_sh.at[pl.ds(r, CHUNK)], cstage_v)

        @pl.when(c == 0)
        def _():
          pltpu.sync_copy(cstage_v, cnt0_out.at[pl.ds(r, CHUNK)])

        @pl.when(c == 1)
        def _():
          pltpu.sync_copy(cstage_v, cnt1_out.at[pl.ds(r, CHUNK)])

  return sc_kernel


_sc_layer1 = _make_sc_layer(True)
_sc_layer2 = _make_sc_layer(False)


_ROW_BLK = 1024


def _dotT(a, w):
  return lax.dot_general(a, w, (((1,), (1,)), ((), ())),
                         precision=lax.Precision.HIGHEST,
                         preferred_element_type=jnp.float32)


def _tc1_body(x_ref, agg_ref, cnt_ref, cnt2_ref, wl_ref, b_ref, wr_ref,
              h_ref):
  inv = 1.0 / jnp.maximum(cnt_ref[:, 0:1] + cnt2_ref[:, 0:1], 1.0)
  mean = agg_ref[...].astype(jnp.float32) * inv
  h = (_dotT(mean, wl_ref[...]) + b_ref[...] +
       _dotT(x_ref[...].astype(jnp.float32), wr_ref[...]))
  h_ref[...] = jnp.maximum(h, 0.0).astype(jnp.bfloat16)


def _tc2_body(h_ref, agg_ref, cnt_ref, cnt2_ref, wl_ref, b_ref, wr_ref,
              wc_ref, bc_ref, out_ref):
  inv = 1.0 / jnp.maximum(cnt_ref[:, 0:1] + cnt2_ref[:, 0:1], 1.0)
  mean = agg_ref[...].astype(jnp.float32) * inv
  h2 = (_dotT(mean, wl_ref[...]) + b_ref[...] +
        _dotT(h_ref[...].astype(jnp.float32), wr_ref[...]))
  out_ref[...] = _dotT(h2, wc_ref[...]) + bc_ref[...]


def _row_spec(width):
  return pl.BlockSpec((_ROW_BLK, width), lambda i: (i, 0))


def _full_spec(r, cdim):
  return pl.BlockSpec((r, cdim), lambda i: (0, 0))


def _tc_layer1(x, agg, cnt0, cnt1, W1_l, b1, W1_r):
  return pl.pallas_call(
      _tc1_body,
      grid=(N_PAD // _ROW_BLK,),
      in_specs=[
          _row_spec(D), _row_spec(D), _row_spec(16), _row_spec(16),
          _full_spec(D, D), _full_spec(1, D), _full_spec(D, D),
      ],
      out_specs=_row_spec(D),
      out_shape=jax.ShapeDtypeStruct((N_PAD, D), jnp.bfloat16),
  )(x, agg, cnt0, cnt1, W1_l, b1.reshape(1, D), W1_r)


def _tc_layer2(h1, agg2, cnt0, cnt1, W2_l, b2, W2_r, Wc, bc):
  return pl.pallas_call(
      _tc2_body,
      grid=(N_PAD // _ROW_BLK,),
      in_specs=[
          _row_spec(D), _row_spec(D), _row_spec(16), _row_spec(16),
          _full_spec(D, D), _full_spec(1, D), _full_spec(D, D),
          _full_spec(NUM_NODE_TYPES, D), _full_spec(1, NUM_NODE_TYPES),
      ],
      out_specs=_row_spec(NUM_NODE_TYPES),
      out_shape=jax.ShapeDtypeStruct((N_PAD, NUM_NODE_TYPES), jnp.float32),
  )(h1, agg2, cnt0, cnt1, W2_l, b2.reshape(1, D), W2_r, Wc,
    bc.reshape(1, NUM_NODE_TYPES))


@jax.jit
def kernel(entity, edge_index, emb_table, W1_l, b1, W1_r, W2_l, b2, W2_r, Wc,
           bc):
  entity_pad = jnp.pad(entity.astype(jnp.int32), (0, N_PAD - N))
  src = edge_index[0].astype(jnp.int32)
  dst = edge_index[1].astype(jnp.int32)
  src_pad = jnp.pad(src, (0, E_PAD - E)).reshape(E_PAD // CHUNK, CHUNK)
  dst_pad = jnp.pad(dst, (0, E_PAD - E),
                    constant_values=N).reshape(E_PAD // CHUNK, CHUNK)
  table_bf = emb_table.astype(jnp.bfloat16)
  zeros = jnp.zeros((STAGE_ROWS, HALF), jnp.bfloat16)
  ones = jnp.concatenate([jnp.zeros((CHUNK, 16), jnp.float32),
                          jnp.ones((CHUNK, 16), jnp.float32)])

  x, agg1, cnt0, cnt1 = _sc_layer1(entity_pad, table_bf, src_pad, dst_pad,
                                   zeros, ones)
  h1 = _tc_layer1(x, agg1, cnt0, cnt1, W1_l, b1, W1_r)
  agg2 = _sc_layer2(h1, src_pad, dst_pad, zeros, ones)
  out = _tc_layer2(h1, agg2, cnt0, cnt1, W2_l, b2, W2_r, Wc, bc)
  return out[:N]


# speedup vs baseline: 12.5771x; 1.4003x over previous
import functools

import jax
import jax.numpy as jnp
from jax import lax
from jax.experimental import pallas as pl
from jax.experimental.pallas import tpu as pltpu
from jax.experimental.pallas import tpu_sc as plsc

N = 10000
E = 320000
NUM_NODE_TYPES = 16

D = 128
HALF = 64
NUM_SUBCORES = 16
CHUNK = 128
BLK = 32
NBLK = 5
N_PAD = 10240
ROWS_PER_TILE = N_PAD // NUM_SUBCORES
EDGES_PER_TILE = NBLK * BLK * CHUNK
E_PAD = EDGES_PER_TILE * NUM_SUBCORES
EMB_CHUNK = 64
EMB_ITERS = ROWS_PER_TILE // EMB_CHUNK
STAGE_ROWS = CHUNK
STAGE_ITERS = ROWS_PER_TILE // STAGE_ROWS


def _make_sc_layer(first_layer: bool):
  mesh = plsc.VectorSubcoreMesh(core_axis_name="c", subcore_axis_name="s")

  if first_layer:
    out_type = (
        jax.ShapeDtypeStruct((N_PAD, D), jnp.float32),
        jax.ShapeDtypeStruct((N_PAD, D), jnp.float32),
        jax.ShapeDtypeStruct((N_PAD, 16), jnp.float32),
        jax.ShapeDtypeStruct((N_PAD, 16), jnp.float32),
    )
  else:
    out_type = jax.ShapeDtypeStruct((N_PAD, D), jnp.float32)

  scratch_types = [
      pltpu.VMEM((EMB_CHUNK,), jnp.int32),
      pltpu.VMEM((EMB_CHUNK,), jnp.int32),
      pltpu.VMEM((EMB_CHUNK, D), jnp.float32),
      pltpu.VMEM((EMB_CHUNK, D), jnp.float32),
      pltpu.VMEM((CHUNK, HALF), jnp.float32),
      pltpu.VMEM((CHUNK, HALF), jnp.float32),
      pltpu.VMEM((CHUNK, HALF), jnp.bfloat16),
      pltpu.VMEM((CHUNK, HALF), jnp.bfloat16),
      pltpu.VMEM((BLK, CHUNK), jnp.int32),
      pltpu.VMEM((BLK, CHUNK), jnp.int32),
      pltpu.VMEM((CHUNK, HALF), jnp.bfloat16),
      pltpu.VMEM((CHUNK, HALF), jnp.bfloat16),
      pltpu.VMEM((CHUNK, HALF), jnp.bfloat16),
      pltpu.VMEM((CHUNK, 16), jnp.float32),
      pltpu.VMEM_SHARED((N_PAD, HALF), jnp.bfloat16),
      pltpu.VMEM_SHARED((N_PAD, HALF), jnp.bfloat16),
      pltpu.VMEM_SHARED((N_PAD, 16), jnp.float32),
      pltpu.SemaphoreType.DMA,
      pltpu.SemaphoreType.DMA,
      pltpu.SemaphoreType.DMA,
      pltpu.SemaphoreType.DMA,
      pltpu.SemaphoreType.DMA,
      pltpu.SemaphoreType.DMA,
      pltpu.SemaphoreType.DMA,
      pltpu.SemaphoreType.DMA,
      pltpu.SemaphoreType.DMA,
  ]

  @functools.partial(
      pl.kernel, out_type=out_type, mesh=mesh, scratch_types=scratch_types,
      name="sc_sage_agg1" if first_layer else "sc_sage_agg2",
      compiler_params=pltpu.CompilerParams(use_tc_tiling_on_sc=False,
                                           needs_layout_passes=False),
  )
  def sc_kernel(*refs):
    if first_layer:
      (entity_hbm, table_hbm, src_hbm, dst_hbm, zeros_hbm, ones_hbm,
       x_out, agg_out, cnt0_out, cnt1_out, *rest) = refs
      h_hbm = None
    else:
      (h_hbm, src_hbm, dst_hbm, zeros_hbm, ones_hbm, agg_out, *rest) = refs
      cnt0_out = cnt1_out = None
    (eidx0, eidx1, emb0, emb1, stg0, stg1, pbf0, pbf1,
     sidx_v, didx_v, msg0_v, msg1_v, msg2_v,
     cstage_v, x_sh, agg_sh, cnt_sh,
     sem, sem_g0, sem_g1, sem_g2, sem_s0, sem_s1, sem_s2, sem_w0,
     sem_w1) = rest
    stgs = (stg0, stg1)
    pbfs = (pbf0, pbf1)

    def pack_rows(f32_ref, col0, bf_ref, nrows):
      def body(i, carry):
        for g in range(HALF // 32):
          a = f32_ref[i, pl.ds(col0 + g * 32, 16)]
          b = f32_ref[i, pl.ds(col0 + g * 32 + 16, 16)]
          bf_ref[i, pl.ds(g * 32, 32)] = plsc.pack(
              a, b, format=plsc.PackFormat.INTERLEAVED)
        return carry
      lax.fori_loop(0, nrows, body, 0)

    def unpack_rows(bf_ref, f32_ref, nrows):
      def body(i, carry):
        for g in range(HALF // 32):
          ab = bf_ref[i, pl.ds(g * 32, 32)]
          a, b = plsc.unpack(ab, format=plsc.PackFormat.INTERLEAVED)
          f32_ref[i, pl.ds(g * 32, 16)] = a
          f32_ref[i, pl.ds(g * 32 + 16, 16)] = b
        return carry
      lax.fori_loop(0, nrows, body, 0)
    eidxs = (eidx0, eidx1)
    embs = (emb0, emb1)
    msgs = (msg0_v, msg1_v, msg2_v)
    gsems = (sem_g0, sem_g1, sem_g2)
    ssems = (sem_s0, sem_s1, sem_s2)
    wsems = (sem_w0, sem_w1)

    c = lax.axis_index("c")
    s = lax.axis_index("s")
    row0 = s * ROWS_PER_TILE
    chalf = c * HALF

    pltpu.sync_copy(zeros_hbm, msgs[0])
    zcp = []
    for k in range(STAGE_ITERS):
      r = row0 + k * STAGE_ROWS
      zcp.append(pltpu.async_copy(msgs[0], agg_sh.at[pl.ds(r, STAGE_ROWS)],
                                  sem_w0))
    if first_layer:
      pltpu.sync_copy(ones_hbm.at[pl.ds(0, CHUNK)], cstage_v)
      for k in range(STAGE_ITERS):
        r = row0 + k * STAGE_ROWS
        pltpu.sync_copy(cstage_v, cnt_sh.at[pl.ds(r, STAGE_ROWS)])
      pltpu.sync_copy(ones_hbm.at[pl.ds(CHUNK, CHUNK)], cstage_v)
    for cp in zcp:
      cp.wait()

    if first_layer:
      gth = [None, None]
      wrs = [[], []]
      pltpu.sync_copy(entity_hbm.at[pl.ds(row0, EMB_CHUNK)], eidxs[0])
      gth[0] = pltpu.async_copy(table_hbm.at[eidxs[0]], embs[0], gsems[0])
      for k in range(EMB_ITERS):
        p = k % 2
        q = (k + 1) % 2
        base = row0 + k * EMB_CHUNK
        if k + 1 < EMB_ITERS:
          for w in wrs[q]:
            w.wait()
          wrs[q] = []
          nbase = base + EMB_CHUNK
          pltpu.sync_copy(entity_hbm.at[pl.ds(nbase, EMB_CHUNK)], eidxs[q])
          gth[q] = pltpu.async_copy(table_hbm.at[eidxs[q]], embs[q], gsems[q])
        gth[p].wait()
        half = embs[p].at[:, pl.ds(chalf, HALF)]
        pack_rows(embs[p], chalf, pbfs[p], EMB_CHUNK)
        wrs[p] = [
            pltpu.async_copy(half,
                             x_out.at[pl.ds(base, EMB_CHUNK),
                                      pl.ds(chalf, HALF)], wsems[p]),
            pltpu.async_copy(pbfs[p].at[pl.ds(0, EMB_CHUNK)],
                             x_sh.at[pl.ds(base, EMB_CHUNK)], ssems[p]),
        ]
      for ws in wrs:
        for w in ws:
          w.wait()
    else:
      stg = [None, None]
      for k in range(STAGE_ITERS):
        p = k % 2
        r = row0 + k * STAGE_ROWS
        if stg[p] is not None:
          stg[p].wait()
        pltpu.sync_copy(
            h_hbm.at[pl.ds(r, STAGE_ROWS), pl.ds(chalf, HALF)], stgs[p])
        pack_rows(stgs[p], 0, pbfs[p], STAGE_ROWS)
        stg[p] = pltpu.async_copy(pbfs[p], x_sh.at[pl.ds(r, STAGE_ROWS)],
                                  ssems[p])
      for st in stg:
        if st is not None:
          st.wait()

    plsc.subcore_barrier()

    def block_body(blk, carry):
      brow = s * (NBLK * BLK) + blk * BLK
      pltpu.sync_copy(src_hbm.at[pl.ds(brow, BLK)], sidx_v)
      pltpu.sync_copy(dst_hbm.at[pl.ds(brow, BLK)], didx_v)
      cnt_copies = []
      gathers = [None, None, None]
      scatters = [None, None, None]
      gathers[0] = pltpu.async_copy(x_sh.at[sidx_v.at[0]], msgs[0], gsems[0])
      gathers[1] = pltpu.async_copy(x_sh.at[sidx_v.at[1]], msgs[1], gsems[1])
      for j in range(BLK):
        p = j % 3
        q = (j + 2) % 3
        if j + 2 < BLK:
          if scatters[q] is not None:
            scatters[q].wait()
            scatters[q] = None
          gathers[q] = pltpu.async_copy(x_sh.at[sidx_v.at[j + 2]], msgs[q],
                                        gsems[q])
        gathers[p].wait()
        scatters[p] = pltpu.async_copy(msgs[p], agg_sh.at[didx_v.at[j]],
                                       ssems[p], add=True)
        if first_layer:
          count_here = (c == 0) if j < BLK // 2 else (c == 1)

          @pl.when(count_here)
          def _():
            cnt_copies.append(
                pltpu.async_copy(cstage_v, cnt_sh.at[didx_v.at[j]], sem,
                                 add=True))
      for sc in scatters:
        if sc is not None:
          sc.wait()
      if first_layer:
        for j, cp in enumerate(cnt_copies):
          count_here = (c == 0) if j < BLK // 2 else (c == 1)

          @pl.when(count_here)
          def _():
            cp.wait()
      return carry

    lax.fori_loop(0, NBLK, block_body, 0)

    plsc.subcore_barrier()

    wr = [None, None]
    for k in range(STAGE_ITERS):
      p = k % 2
      r = row0 + k * STAGE_ROWS
      if wr[p] is not None:
        wr[p].wait()
      pltpu.sync_copy(agg_sh.at[pl.ds(r, STAGE_ROWS)], pbfs[p])
      unpack_rows(pbfs[p], stgs[p], STAGE_ROWS)
      wr[p] = pltpu.async_copy(
          stgs[p], agg_out.at[pl.ds(r, STAGE_ROWS), pl.ds(chalf, HALF)],
          wsems[p])
    for w in wr:
      if w is not None:
        w.wait()
    if first_layer:
      for k in range(ROWS_PER_TILE // CHUNK):
        r = row0 + k * CHUNK
        pltpu.sync_copy(cnt_sh.at[pl.ds(r, CHUNK)], cstage_v)

        @pl.when(c == 0)
        def _():
          pltpu.sync_copy(cstage_v, cnt0_out.at[pl.ds(r, CHUNK)])

        @pl.when(c == 1)
        def _():
          pltpu.sync_copy(cstage_v, cnt1_out.at[pl.ds(r, CHUNK)])

  return sc_kernel


_sc_layer1 = _make_sc_layer(True)
_sc_layer2 = _make_sc_layer(False)


_ROW_BLK = 1024


def _dotT(a, w):
  return lax.dot_general(a, w, (((1,), (1,)), ((), ())),
                         preferred_element_type=jnp.float32)


def _tc1_body(x_ref, agg_ref, cnt_ref, cnt2_ref, wl_ref, b_ref, wr_ref,
              h_ref):
  inv = 1.0 / jnp.maximum(cnt_ref[:, 0:1] + cnt2_ref[:, 0:1], 1.0)
  mean = agg_ref[...] * inv
  h = (_dotT(mean, wl_ref[...]) + b_ref[...] +
       _dotT(x_ref[...], wr_ref[...]))
  h_ref[...] = jnp.maximum(h, 0.0)


def _tc2_body(h_ref, agg_ref, cnt_ref, cnt2_ref, wl_ref, b_ref, wr_ref,
              wc_ref, bc_ref, out_ref):
  inv = 1.0 / jnp.maximum(cnt_ref[:, 0:1] + cnt2_ref[:, 0:1], 1.0)
  mean = agg_ref[...] * inv
  h2 = (_dotT(mean, wl_ref[...]) + b_ref[...] +
        _dotT(h_ref[...], wr_ref[...]))
  out_ref[...] = _dotT(h2, wc_ref[...]) + bc_ref[...]


def _row_spec(width):
  return pl.BlockSpec((_ROW_BLK, width), lambda i: (i, 0))


def _full_spec(r, cdim):
  return pl.BlockSpec((r, cdim), lambda i: (0, 0))


def _tc_layer1(x, agg, cnt0, cnt1, W1_l, b1, W1_r):
  return pl.pallas_call(
      _tc1_body,
      grid=(N_PAD // _ROW_BLK,),
      in_specs=[
          _row_spec(D), _row_spec(D), _row_spec(16), _row_spec(16),
          _full_spec(D, D), _full_spec(1, D), _full_spec(D, D),
      ],
      out_specs=_row_spec(D),
      out_shape=jax.ShapeDtypeStruct((N_PAD, D), jnp.float32),
  )(x, agg, cnt0, cnt1, W1_l, b1.reshape(1, D), W1_r)


def _tc_layer2(h1, agg2, cnt0, cnt1, W2_l, b2, W2_r, Wc, bc):
  return pl.pallas_call(
      _tc2_body,
      grid=(N_PAD // _ROW_BLK,),
      in_specs=[
          _row_spec(D), _row_spec(D), _row_spec(16), _row_spec(16),
          _full_spec(D, D), _full_spec(1, D), _full_spec(D, D),
          _full_spec(NUM_NODE_TYPES, D), _full_spec(1, NUM_NODE_TYPES),
      ],
      out_specs=_row_spec(NUM_NODE_TYPES),
      out_shape=jax.ShapeDtypeStruct((N_PAD, NUM_NODE_TYPES), jnp.float32),
  )(h1, agg2, cnt0, cnt1, W2_l, b2.reshape(1, D), W2_r, Wc,
    bc.reshape(1, NUM_NODE_TYPES))


@jax.jit
def kernel(entity, edge_index, emb_table, W1_l, b1, W1_r, W2_l, b2, W2_r, Wc,
           bc):
  entity_pad = jnp.pad(entity.astype(jnp.int32), (0, N_PAD - N))
  src = edge_index[0].astype(jnp.int32)
  dst = edge_index[1].astype(jnp.int32)
  src_pad = jnp.pad(src, (0, E_PAD - E)).reshape(E_PAD // CHUNK, CHUNK)
  dst_pad = jnp.pad(dst, (0, E_PAD - E),
                    constant_values=N).reshape(E_PAD // CHUNK, CHUNK)
  zeros = jnp.zeros((STAGE_ROWS, HALF), jnp.bfloat16)
  ones = jnp.concatenate([jnp.zeros((CHUNK, 16), jnp.float32),
                          jnp.ones((CHUNK, 16), jnp.float32)])

  x, agg1, cnt0, cnt1 = _sc_layer1(entity_pad, emb_table, src_pad, dst_pad,
                                   zeros, ones)
  h1 = _tc_layer1(x, agg1, cnt0, cnt1, W1_l, b1, W1_r)
  agg2 = _sc_layer2(h1, src_pad, dst_pad, zeros, ones)
  out = _tc_layer2(h1, agg2, cnt0, cnt1, W2_l, b2, W2_r, Wc, bc)
  return out[:N]


# speedup vs baseline: 12.5858x; 1.0007x over previous
import functools

import jax
import jax.numpy as jnp
from jax import lax
from jax.experimental import pallas as pl
from jax.experimental.pallas import tpu as pltpu
from jax.experimental.pallas import tpu_sc as plsc

N = 10000
E = 320000
NUM_NODE_TYPES = 16

D = 128
HALF = 64
NUM_SUBCORES = 16
CHUNK = 128
BLK = 32
NBLK = 5
N_PAD = 10240
ROWS_PER_TILE = N_PAD // NUM_SUBCORES
EDGES_PER_TILE = NBLK * BLK * CHUNK
E_PAD = EDGES_PER_TILE * NUM_SUBCORES
EMB_CHUNK = 64
EMB_ITERS = ROWS_PER_TILE // EMB_CHUNK
STAGE_ROWS = CHUNK
STAGE_ITERS = ROWS_PER_TILE // STAGE_ROWS


def _make_sc_layer(first_layer: bool):
  mesh = plsc.VectorSubcoreMesh(core_axis_name="c", subcore_axis_name="s")

  if first_layer:
    out_type = (
        jax.ShapeDtypeStruct((N_PAD, D), jnp.float32),
        jax.ShapeDtypeStruct((N_PAD, D), jnp.float32),
        jax.ShapeDtypeStruct((N_PAD, 16), jnp.float32),
        jax.ShapeDtypeStruct((N_PAD, 16), jnp.float32),
    )
  else:
    out_type = jax.ShapeDtypeStruct((N_PAD, D), jnp.float32)

  scratch_types = [
      pltpu.VMEM((EMB_CHUNK,), jnp.int32),
      pltpu.VMEM((EMB_CHUNK,), jnp.int32),
      pltpu.VMEM((EMB_CHUNK, D), jnp.float32),
      pltpu.VMEM((EMB_CHUNK, D), jnp.float32),
      pltpu.VMEM((CHUNK, HALF), jnp.float32),
      pltpu.VMEM((CHUNK, HALF), jnp.float32),
      pltpu.VMEM((CHUNK, HALF), jnp.bfloat16),
      pltpu.VMEM((CHUNK, HALF), jnp.bfloat16),
      pltpu.VMEM((BLK, CHUNK), jnp.int32),
      pltpu.VMEM((BLK, CHUNK), jnp.int32),
      pltpu.VMEM((CHUNK, HALF), jnp.bfloat16),
      pltpu.VMEM((CHUNK, HALF), jnp.bfloat16),
      pltpu.VMEM((CHUNK, HALF), jnp.bfloat16),
      pltpu.VMEM((CHUNK, 16), jnp.float32),
      pltpu.VMEM_SHARED((N_PAD, HALF), jnp.bfloat16),
      pltpu.VMEM_SHARED((N_PAD, HALF), jnp.bfloat16),
      pltpu.VMEM_SHARED((N_PAD, 16), jnp.float32),
      pltpu.SemaphoreType.DMA,
      pltpu.SemaphoreType.DMA,
      pltpu.SemaphoreType.DMA,
      pltpu.SemaphoreType.DMA,
      pltpu.SemaphoreType.DMA,
      pltpu.SemaphoreType.DMA,
      pltpu.SemaphoreType.DMA,
      pltpu.SemaphoreType.DMA,
      pltpu.SemaphoreType.DMA,
  ]

  @functools.partial(
      pl.kernel, out_type=out_type, mesh=mesh, scratch_types=scratch_types,
      name="sc_sage_agg1" if first_layer else "sc_sage_agg2",
      compiler_params=pltpu.CompilerParams(use_tc_tiling_on_sc=False,
                                           needs_layout_passes=False),
  )
  def sc_kernel(*refs):
    if first_layer:
      (entity_hbm, table_hbm, src_hbm, dst_hbm, zeros_hbm, ones_hbm,
       x_out, agg_out, cnt0_out, cnt1_out, *rest) = refs
      h_hbm = None
    else:
      (h_hbm, src_hbm, dst_hbm, zeros_hbm, ones_hbm, agg_out, *rest) = refs
      cnt0_out = cnt1_out = None
    (eidx0, eidx1, emb0, emb1, stg0, stg1, pbf0, pbf1,
     sidx_v, didx_v, msg0_v, msg1_v, msg2_v,
     cstage_v, x_sh, agg_sh, cnt_sh,
     sem, sem_g0, sem_g1, sem_g2, sem_s0, sem_s1, sem_s2, sem_w0,
     sem_w1) = rest
    stgs = (stg0, stg1)
    pbfs = (pbf0, pbf1)

    def pack_rows(f32_ref, col0, bf_ref, nrows):
      def body(i, carry):
        for g in range(HALF // 32):
          a = f32_ref[i, pl.ds(col0 + g * 32, 16)]
          b = f32_ref[i, pl.ds(col0 + g * 32 + 16, 16)]
          bf_ref[i, pl.ds(g * 32, 32)] = plsc.pack(
              a, b, format=plsc.PackFormat.INTERLEAVED)
        return carry
      lax.fori_loop(0, nrows, body, 0)

    def unpack_rows(bf_ref, f32_ref, nrows):
      def body(i, carry):
        for g in range(HALF // 32):
          ab = bf_ref[i, pl.ds(g * 32, 32)]
          a, b = plsc.unpack(ab, format=plsc.PackFormat.INTERLEAVED)
          f32_ref[i, pl.ds(g * 32, 16)] = a
          f32_ref[i, pl.ds(g * 32 + 16, 16)] = b
        return carry
      lax.fori_loop(0, nrows, body, 0)
    eidxs = (eidx0, eidx1)
    embs = (emb0, emb1)
    msgs = (msg0_v, msg1_v, msg2_v)
    gsems = (sem_g0, sem_g1, sem_g2)
    ssems = (sem_s0, sem_s1, sem_s2)
    wsems = (sem_w0, sem_w1)

    c = lax.axis_index("c")
    s = lax.axis_index("s")
    row0 = s * ROWS_PER_TILE
    chalf = c * HALF

    pltpu.sync_copy(zeros_hbm, msgs[0])
    zcp = []
    for k in range(STAGE_ITERS):
      r = row0 + k * STAGE_ROWS
      zcp.append(pltpu.async_copy(msgs[0], agg_sh.at[pl.ds(r, STAGE_ROWS)],
                                  sem_w0))
    if first_layer:
      pltpu.sync_copy(ones_hbm.at[pl.ds(0, CHUNK)], cstage_v)
      for k in range(STAGE_ITERS):
        r = row0 + k * STAGE_ROWS
        pltpu.sync_copy(cstage_v, cnt_sh.at[pl.ds(r, STAGE_ROWS)])
      pltpu.sync_copy(ones_hbm.at[pl.ds(CHUNK, CHUNK)], cstage_v)
    for cp in zcp:
      cp.wait()

    if first_layer:
      gth = [None, None]
      wrs = [[], []]
      pltpu.sync_copy(entity_hbm.at[pl.ds(row0, EMB_CHUNK)], eidxs[0])
      gth[0] = pltpu.async_copy(table_hbm.at[eidxs[0]], embs[0], gsems[0])
      for k in range(EMB_ITERS):
        p = k % 2
        q = (k + 1) % 2
        base = row0 + k * EMB_CHUNK
        if k + 1 < EMB_ITERS:
          for w in wrs[q]:
            w.wait()
          wrs[q] = []
          nbase = base + EMB_CHUNK
          pltpu.sync_copy(entity_hbm.at[pl.ds(nbase, EMB_CHUNK)], eidxs[q])
          gth[q] = pltpu.async_copy(table_hbm.at[eidxs[q]], embs[q], gsems[q])
        gth[p].wait()
        half = embs[p].at[:, pl.ds(chalf, HALF)]
        pack_rows(embs[p], chalf, pbfs[p], EMB_CHUNK)
        wrs[p] = [
            pltpu.async_copy(half,
                             x_out.at[pl.ds(base, EMB_CHUNK),
                                      pl.ds(chalf, HALF)], wsems[p]),
            pltpu.async_copy(pbfs[p].at[pl.ds(0, EMB_CHUNK)],
                             x_sh.at[pl.ds(base, EMB_CHUNK)], ssems[p]),
        ]
      for ws in wrs:
        for w in ws:
          w.wait()
    else:
      stg = [None, None]
      for k in range(STAGE_ITERS):
        p = k % 2
        r = row0 + k * STAGE_ROWS
        if stg[p] is not None:
          stg[p].wait()
        pltpu.sync_copy(
            h_hbm.at[pl.ds(r, STAGE_ROWS), pl.ds(chalf, HALF)], stgs[p])
        pack_rows(stgs[p], 0, pbfs[p], STAGE_ROWS)
        stg[p] = pltpu.async_copy(pbfs[p], x_sh.at[pl.ds(r, STAGE_ROWS)],
                                  ssems[p])
      for st in stg:
        if st is not None:
          st.wait()

    plsc.subcore_barrier()

    def block_body(blk, carry):
      brow = s * (NBLK * BLK) + blk * BLK
      pltpu.sync_copy(src_hbm.at[pl.ds(brow, BLK)], sidx_v)
      pltpu.sync_copy(dst_hbm.at[pl.ds(brow, BLK)], didx_v)
      cnt_copies = []
      gathers = [None, None, None]
      scatters = [None, None, None]
      gathers[0] = pltpu.async_copy(x_sh.at[sidx_v.at[0]], msgs[0], gsems[0])
      gathers[1] = pltpu.async_copy(x_sh.at[sidx_v.at[1]], msgs[1], gsems[1])
      for j in range(BLK):
        p = j % 3
        q = (j + 2) % 3
        if j + 2 < BLK:
          if scatters[q] is not None:
            scatters[q].wait()
            scatters[q] = None
          gathers[q] = pltpu.async_copy(x_sh.at[sidx_v.at[j + 2]], msgs[q],
                                        gsems[q])
        gathers[p].wait()
        scatters[p] = pltpu.async_copy(msgs[p], agg_sh.at[didx_v.at[j]],
                                       ssems[p], add=True)
        if first_layer:
          count_here = (c == 0) if j < BLK // 2 else (c == 1)

          @pl.when(count_here)
          def _():
            cnt_copies.append(
                pltpu.async_copy(cstage_v, cnt_sh.at[didx_v.at[j]], sem,
                                 add=True))
      for sc in scatters:
        if sc is not None:
          sc.wait()
      if first_layer:
        for j, cp in enumerate(cnt_copies):
          count_here = (c == 0) if j < BLK // 2 else (c == 1)

          @pl.when(count_here)
          def _():
            cp.wait()
      return carry

    lax.fori_loop(0, NBLK, block_body, 0)

    plsc.subcore_barrier()

    wr = [None, None]
    for k in range(STAGE_ITERS):
      p = k % 2
      r = row0 + k * STAGE_ROWS
      if wr[p] is not None:
        wr[p].wait()
      pltpu.sync_copy(agg_sh.at[pl.ds(r, STAGE_ROWS)], pbfs[p])
      unpack_rows(pbfs[p], stgs[p], STAGE_ROWS)
      wr[p] = pltpu.async_copy(
          stgs[p], agg_out.at[pl.ds(r, STAGE_ROWS), pl.ds(chalf, HALF)],
          wsems[p])
    for w in wr:
      if w is not None:
        w.wait()
    if first_layer:
      for k in range(ROWS_PER_TILE // CHUNK):
        r = row0 + k * CHUNK
        pltpu.sync_copy(cnt_sh.at[pl.ds(r, CHUNK)], cstage_v)

        @pl.when(c == 0)
        def _():
          pltpu.sync_copy(cstage_v, cnt0_out.at[pl.ds(r, CHUNK)])

        @pl.when(c == 1)
        def _():
          pltpu.sync_copy(cstage_v, cnt1_out.at[pl.ds(r, CHUNK)])

  return sc_kernel


_sc_layer1 = _make_sc_layer(True)
_sc_layer2 = _make_sc_layer(False)


_ROW_BLK = 1024


def _dotT(a, w):
  return lax.dot_general(a, w, (((1,), (1,)), ((), ())),
                         preferred_element_type=jnp.float32)


def _tc1_body(x_ref, agg_ref, cnt_ref, cnt2_ref, wl_ref, b_ref, wr_ref,
              h_ref):
  inv = 1.0 / jnp.maximum(cnt_ref[:, 0:1] + cnt2_ref[:, 0:1], 1.0)
  mean = agg_ref[...] * inv
  h = (_dotT(mean, wl_ref[...]) + b_ref[...] +
       _dotT(x_ref[...], wr_ref[...]))
  h_ref[...] = jnp.maximum(h, 0.0)


def _tc2_body(h_ref, agg_ref, cnt_ref, cnt2_ref, wl_ref, b_ref, wr_ref,
              wc_ref, bc_ref, out_ref):
  inv = 1.0 / jnp.maximum(cnt_ref[:, 0:1] + cnt2_ref[:, 0:1], 1.0)
  mean = agg_ref[...] * inv
  h2 = (_dotT(mean, wl_ref[...]) + b_ref[...] +
        _dotT(h_ref[...], wr_ref[...]))
  out_ref[...] = _dotT(h2, wc_ref[...]) + bc_ref[...]


def _row_spec(width):
  return pl.BlockSpec((_ROW_BLK, width), lambda i: (i, 0))


def _full_spec(r, cdim):
  return pl.BlockSpec((r, cdim), lambda i: (0, 0))


def _tc_layer1(x, agg, cnt0, cnt1, W1_l, b1, W1_r):
  return pl.pallas_call(
      _tc1_body,
      grid=(N_PAD // _ROW_BLK,),
      in_specs=[
          _row_spec(D), _row_spec(D), _row_spec(16), _row_spec(16),
          _full_spec(D, D), _full_spec(1, D), _full_spec(D, D),
      ],
      out_specs=_row_spec(D),
      out_shape=jax.ShapeDtypeStruct((N_PAD, D), jnp.float32),
  )(x, agg, cnt0, cnt1, W1_l, b1.reshape(1, D), W1_r)


def _tc_layer2(h1, agg2, cnt0, cnt1, W2_l, b2, W2_r, Wc, bc):
  return pl.pallas_call(
      _tc2_body,
      grid=(N_PAD // _ROW_BLK,),
      in_specs=[
          _row_spec(D), _row_spec(D), _row_spec(16), _row_spec(16),
          _full_spec(D, D), _full_spec(1, D), _full_spec(D, D),
          _full_spec(NUM_NODE_TYPES, D), _full_spec(1, NUM_NODE_TYPES),
      ],
      out_specs=_row_spec(NUM_NODE_TYPES),
      out_shape=jax.ShapeDtypeStruct((N, NUM_NODE_TYPES), jnp.float32),
  )(h1, agg2, cnt0, cnt1, W2_l, b2.reshape(1, D), W2_r, Wc,
    bc.reshape(1, NUM_NODE_TYPES))


@jax.jit
def kernel(entity, edge_index, emb_table, W1_l, b1, W1_r, W2_l, b2, W2_r, Wc,
           bc):
  entity_pad = jnp.pad(entity.astype(jnp.int32), (0, N_PAD - N))
  src = edge_index[0].astype(jnp.int32)
  dst = edge_index[1].astype(jnp.int32)
  src_pad = jnp.pad(src, (0, E_PAD - E)).reshape(E_PAD // CHUNK, CHUNK)
  dst_pad = jnp.pad(dst, (0, E_PAD - E),
                    constant_values=N).reshape(E_PAD // CHUNK, CHUNK)
  zeros = jnp.zeros((STAGE_ROWS, HALF), jnp.bfloat16)
  ones = jnp.concatenate([jnp.zeros((CHUNK, 16), jnp.float32),
                          jnp.ones((CHUNK, 16), jnp.float32)])

  x, agg1, cnt0, cnt1 = _sc_layer1(entity_pad, emb_table, src_pad, dst_pad,
                                   zeros, ones)
  h1 = _tc_layer1(x, agg1, cnt0, cnt1, W1_l, b1, W1_r)
  agg2 = _sc_layer2(h1, src_pad, dst_pad, zeros, ones)
  return _tc_layer2(h1, agg2, cnt0, cnt1, W2_l, b2, W2_r, Wc, bc)
